# baseline (speedup 1.0000x reference)
import functools

import jax
import jax.numpy as jnp
from jax import lax
from jax.experimental import pallas as pl
from jax.experimental.pallas import tpu as pltpu
from jax.experimental.pallas import tpu_sc as plsc

N = 10000
E = 320000
D = 128

NC = 2
NS = 16
NW = NC * NS
C = 128
NBUF = 2
K = 80
EPAD = NW * K * C
NPAD = 10112
RZ = NPAD // NS


NIB = 4


def _segment_sum_sc(x, idx, zeros):
  mesh = plsc.VectorSubcoreMesh(core_axis_name="c", subcore_axis_name="s")

  @functools.partial(
      pl.kernel,
      mesh=mesh,
      out_type=jax.ShapeDtypeStruct((NC, NPAD, D), jnp.float32),
      scratch_types=[
          pltpu.VMEM((NIB, 2, C), jnp.int32),
          pltpu.VMEM((C, D), jnp.float32),
          pltpu.VMEM((C, D), jnp.float32),
          pltpu.VMEM_SHARED((NPAD, D), jnp.float32),
          [pltpu.SemaphoreType.DMA] * NIB,
          [pltpu.SemaphoreType.DMA] * NBUF,
      ],
  )
  def seg_kernel(x_hbm, idx_hbm, zero_hbm, out_hbm,
                 ib_v, rows0_v, rows1_v, acc_sh, isems, gsems):
    cid = lax.axis_index("c")
    sid = lax.axis_index("s")
    wid = sid * NC + cid

    pltpu.sync_copy(zero_hbm.at[pl.ds(sid * RZ, RZ)],
                    acc_sh.at[pl.ds(sid * RZ, RZ)])
    plsc.subcore_barrier()

    rows = (rows0_v, rows1_v)
    my_idx = idx_hbm.at[wid]

    for j in range(NIB):
      pltpu.async_copy(my_idx.at[j], ib_v.at[j], isems[j])
    pltpu.make_async_copy(my_idx.at[0], ib_v.at[0], isems[0]).wait()
    pltpu.async_copy(x_hbm.at[ib_v.at[0].at[0]], rows[0], gsems[0])

    @pl.loop(0, K, step=NIB)
    def _(k):
      for j in range(NIB):
        kb = k + j
        jn = (j + 1) % NIB
        b = j % NBUF
        nb = (j + 1) % NBUF

        @pl.when(kb + 1 < K)
        def _():
          pltpu.make_async_copy(my_idx.at[jn], ib_v.at[jn], isems[jn]).wait()
          pltpu.async_copy(x_hbm.at[ib_v.at[jn].at[0]], rows[nb], gsems[nb])

        pltpu.make_async_copy(x_hbm.at[ib_v.at[j].at[0]], rows[b],
                              gsems[b]).wait()
        pltpu.sync_copy(rows[b], acc_sh.at[ib_v.at[j].at[1]], add=True)

        @pl.when(kb + NIB < K)
        def _():
          pltpu.async_copy(my_idx.at[kb + NIB], ib_v.at[j], isems[j])

    plsc.subcore_barrier()
    pltpu.sync_copy(acc_sh.at[pl.ds(sid * RZ, RZ)],
                    out_hbm.at[cid].at[pl.ds(sid * RZ, RZ)])

  return seg_kernel(x, idx, zeros)


BN = 1000


def _combine_tc(parts, x, w_rel, b_rel, w_root):
  def body(p0_ref, p1_ref, x_ref, wrel_ref, wroot_ref, b_ref, o_ref):
    agg = p0_ref[0] + p1_ref[0]
    dn = (((1,), (1,)), ((), ()))
    rel = lax.dot_general(agg, wrel_ref[...], dn,
                          preferred_element_type=jnp.float32)
    root = lax.dot_general(x_ref[...], wroot_ref[...], dn,
                           preferred_element_type=jnp.float32)
    o_ref[...] = rel + root + b_ref[...]

  return pl.pallas_call(
      body,
      grid=(N // BN,),
      in_specs=[
          pl.BlockSpec((1, BN, D), lambda i: (0, i, 0)),
          pl.BlockSpec((1, BN, D), lambda i: (1, i, 0)),
          pl.BlockSpec((BN, D), lambda i: (i, 0)),
          pl.BlockSpec((D, D), lambda i: (0, 0)),
          pl.BlockSpec((D, D), lambda i: (0, 0)),
          pl.BlockSpec((D,), lambda i: (0,)),
      ],
      out_specs=pl.BlockSpec((BN, D), lambda i: (i, 0)),
      out_shape=jax.ShapeDtypeStruct((N, D), jnp.float32),
  )(parts, parts, x, w_rel, w_root, b_rel)


def kernel(x, edge_index, W1_rel, b1_rel, W1_root, W2_rel, b2_rel, W2_root):
  src = edge_index[0]
  dst = edge_index[1]
  pad = EPAD - E
  srcs = jnp.concatenate([src, jnp.zeros((pad,), jnp.int32)])
  dsts = jnp.concatenate([dst, jnp.full((pad,), N, jnp.int32)])
  idx = jnp.stack([srcs.reshape(NW, K, C), dsts.reshape(NW, K, C)], axis=2)
  zeros = jnp.zeros((NPAD, D), jnp.float32)

  p1 = _segment_sum_sc(x, idx, zeros)
  h = _combine_tc(p1, x, W1_rel, b1_rel, W1_root)
  p2 = _segment_sum_sc(h, idx, zeros)
  return _combine_tc(p2, h, W2_rel, b2_rel, W2_root)
